# baseline (speedup 1.0000x reference)
import functools

import jax
import jax.numpy as jnp
from jax import lax
from jax.experimental import pallas as pl
from jax.experimental.pallas import tpu as pltpu
from jax.experimental.pallas import tpu_sc as plsc

_NC = 2
_NS = 16
_NW = _NC * _NS
_ROW = 67
_SUB = 8
_K = 16


@functools.lru_cache(maxsize=None)
def _make_gather(batch):
    b_per_w = batch // _NW
    n_groups = b_per_w // _K
    mesh = plsc.VectorSubcoreMesh(core_axis_name="c", subcore_axis_name="s")

    @functools.partial(
        pl.kernel,
        mesh=mesh,
        compiler_params=pltpu.CompilerParams(needs_layout_passes=False),
        out_type=jax.ShapeDtypeStruct((batch * _ROW,), jnp.float32),
        scratch_types=[
            pltpu.VMEM((b_per_w,), jnp.int32),
            pltpu.VMEM((b_per_w,), jnp.int32),
            pltpu.VMEM((2 * _K, _SUB, _ROW), jnp.float32),
            pltpu.VMEM((b_per_w * _ROW,), jnp.float32),
            pltpu.SemaphoreType.DMA,
        ],
    )
    def gather_kernel(mem_hbm, cid_hbm, sub_hbm, out_hbm,
                      cid_v, sub_v, chunk_v, rows_v, sem):
        wid = lax.axis_index("s") * _NC + lax.axis_index("c")
        base = wid * b_per_w
        pltpu.sync_copy(cid_hbm.at[pl.ds(base, b_per_w)], cid_v)
        pltpu.sync_copy(sub_hbm.at[pl.ds(base, b_per_w)], sub_v)
        lanes = lax.iota(jnp.int32, 16)

        def fire(g, slot):
            j0 = g * _K
            for b in range(_K // 16):
                vec = cid_v[pl.ds(j0 + b * 16, 16)]
                for k in range(16):
                    pltpu.async_copy(
                        mem_hbm.at[pl.ds(vec[k], 1)],
                        chunk_v.at[pl.ds(slot * _K + b * 16 + k, 1)],
                        sem,
                    )

        def drain_extract(g, slot):
            for _ in range(_K):
                pltpu.make_async_copy(
                    mem_hbm.at[pl.ds(0, 1)],
                    chunk_v.at[pl.ds(0, 1)],
                    sem,
                ).wait()
            j0 = g * _K
            for b in range(_K // 16):
                j_vec = j0 + b * 16 + lanes
                i_vec = slot * _K + b * 16 + lanes
                s_vec = plsc.load_gather(sub_v, [j_vec])
                for c in range(_ROW):
                    c_vec = jnp.full((16,), c, dtype=jnp.int32)
                    vals = plsc.load_gather(chunk_v, [i_vec, s_vec, c_vec])
                    plsc.store_scatter(rows_v, [j_vec * _ROW + c_vec], vals)

        fire(0, 0)

        def do_group(g):
            @pl.when(g < n_groups - 1)
            def _():
                fire(g + 1, (g + 1) % 2)
            drain_extract(g, g % 2)

        pl.loop(0, n_groups)(do_group)
        pltpu.sync_copy(
            rows_v, out_hbm.at[pl.ds(base * _ROW, b_per_w * _ROW)]
        )

    return gather_kernel


def kernel(memory, indices):
    batch = indices.shape[0]
    cap = memory.shape[0]
    mem3 = memory.reshape(cap // _SUB, _SUB, _ROW)
    flat = _make_gather(batch)(mem3, indices // _SUB, indices % _SUB)
    return flat.reshape(batch, _ROW)


# speedup vs baseline: 2.0738x; 2.0738x over previous
import functools

import jax
import jax.numpy as jnp
from jax import lax
from jax.experimental import pallas as pl
from jax.experimental.pallas import tpu as pltpu
from jax.experimental.pallas import tpu_sc as plsc

_NC = 2
_NS = 16
_NW = _NC * _NS
_ROW = 67
_K = 32
_KT = 16


@functools.lru_cache(maxsize=None)
def _make_sc_gather(batch):
    b_per_w = batch // _NW
    n_groups = b_per_w // _K
    mesh = plsc.VectorSubcoreMesh(core_axis_name="c", subcore_axis_name="s")

    @functools.partial(
        pl.kernel,
        mesh=mesh,
        compiler_params=pltpu.CompilerParams(needs_layout_passes=False),
        out_type=jax.ShapeDtypeStruct((batch, _ROW), jnp.float32),
        scratch_types=[
            pltpu.VMEM((b_per_w,), jnp.int32),
            pltpu.VMEM((b_per_w, _ROW), jnp.float32),
            pltpu.SemaphoreType.DMA,
        ],
    )
    def sc_kernel(mem_hbm, idx_hbm, out_hbm, idx_v, rows_v, sem):
        wid = lax.axis_index("s") * _NC + lax.axis_index("c")
        base = wid * b_per_w
        pltpu.sync_copy(idx_hbm.at[pl.ds(base, b_per_w)], idx_v)

        def drain(n):
            for _ in range(n):
                pltpu.make_async_copy(
                    mem_hbm.at[pl.ds(0, 1)], rows_v.at[pl.ds(0, 1)], sem
                ).wait()

        def do_group(g):
            j0 = g * _K
            for b in range(_K // 16):
                vec = idx_v[pl.ds(j0 + b * 16, 16)]
                for k in range(16):
                    pltpu.async_copy(
                        mem_hbm.at[pl.ds(vec[k], 1)],
                        rows_v.at[pl.ds(j0 + b * 16 + k, 1)],
                        sem,
                    )
            @pl.when(g > 0)
            def _():
                drain(_K)

        pl.loop(0, n_groups)(do_group)
        drain(_K)
        pltpu.sync_copy(rows_v, out_hbm.at[pl.ds(base, b_per_w)])

    return sc_kernel


@functools.lru_cache(maxsize=None)
def _make_tc_gather(batch):
    n_groups = batch // _KT

    def tc_kernel(idx_s, mem_hbm, out_hbm, sem):
        def drain(n):
            for _ in range(n):
                pltpu.make_async_copy(
                    mem_hbm.at[pl.ds(0, 1)], out_hbm.at[pl.ds(0, 1)], sem
                ).wait()

        def do_group(g):
            j0 = g * _KT
            for k in range(_KT):
                pltpu.async_copy(
                    mem_hbm.at[pl.ds(idx_s[j0 + k], 1)],
                    out_hbm.at[pl.ds(j0 + k, 1)],
                    sem,
                )
            @pl.when(g > 0)
            def _():
                drain(_KT)

        pl.loop(0, n_groups)(do_group)
        drain(_KT)

    grid_spec = pltpu.PrefetchScalarGridSpec(
        num_scalar_prefetch=1,
        grid=(1,),
        in_specs=[pl.BlockSpec(memory_space=pltpu.HBM)],
        out_specs=pl.BlockSpec(memory_space=pltpu.HBM),
        scratch_shapes=[pltpu.SemaphoreType.DMA],
    )
    return pl.pallas_call(
        tc_kernel,
        grid_spec=grid_spec,
        out_shape=jax.ShapeDtypeStruct((batch, _ROW), jnp.float32),
    )


def kernel(memory, indices):
    batch = indices.shape[0]
    n_sc = (batch * 3 // 4) // (_NW * _K) * (_NW * _K)
    sc_out = _make_sc_gather(n_sc)(memory, indices[:n_sc])
    tc_out = _make_tc_gather(batch - n_sc)(indices[n_sc:], memory)
    return jnp.concatenate([sc_out, tc_out], axis=0)


# speedup vs baseline: 2.9602x; 1.4275x over previous
import functools

import jax
import jax.numpy as jnp
from jax import lax
from jax.experimental import pallas as pl
from jax.experimental.pallas import tpu as pltpu
from jax.experimental.pallas import tpu_sc as plsc

_NC = 2
_NS = 16
_NW = _NC * _NS
_ROW = 67
_K = 32


@functools.lru_cache(maxsize=None)
def _make_gather(batch):
    b_per_w = batch // _NW
    n_groups = b_per_w // _K
    mesh = plsc.VectorSubcoreMesh(core_axis_name="c", subcore_axis_name="s")

    @functools.partial(
        pl.kernel,
        mesh=mesh,
        compiler_params=pltpu.CompilerParams(needs_layout_passes=False),
        out_type=jax.ShapeDtypeStruct((batch, _ROW), jnp.float32),
        scratch_types=[
            pltpu.VMEM((b_per_w,), jnp.int32),
            pltpu.VMEM((b_per_w, _ROW), jnp.float32),
            pltpu.SemaphoreType.DMA,
            pltpu.SemaphoreType.DMA,
            pltpu.SemaphoreType.DMA,
            pltpu.SemaphoreType.DMA,
        ],
    )
    def gather_kernel(mem_hbm, idx_hbm, out_hbm, idx_v, rows_v,
                      sem, sem1, sem2, sem3):
        sems = (sem, sem1, sem2, sem3)
        wid = lax.axis_index("s") * _NC + lax.axis_index("c")
        base = wid * b_per_w
        pltpu.sync_copy(idx_hbm.at[pl.ds(base, b_per_w)], idx_v)

        def drain(n):
            for i in range(n):
                pltpu.make_async_copy(
                    mem_hbm.at[pl.ds(0, 1)], rows_v.at[pl.ds(0, 1)],
                    sems[i % 4],
                ).wait()

        def do_group(g):
            j0 = g * _K
            for b in range(_K // 16):
                vec = idx_v[pl.ds(j0 + b * 16, 16)]
                for k in range(16):
                    pltpu.async_copy(
                        mem_hbm.at[pl.ds(vec[k], 1)],
                        rows_v.at[pl.ds(j0 + b * 16 + k, 1)],
                        sems[k % 4],
                    )
            @pl.when(g > 0)
            def _():
                drain(_K)

        pl.loop(0, n_groups)(do_group)
        drain(_K)
        pltpu.sync_copy(rows_v, out_hbm.at[pl.ds(base, b_per_w)])

    return gather_kernel


def kernel(memory, indices):
    return _make_gather(indices.shape[0])(memory, indices)
